# baseline (speedup 1.0000x reference)
import functools

import jax
import jax.numpy as jnp
from jax import lax
from jax.experimental import pallas as pl
from jax.experimental.pallas import tpu as pltpu
from jax.experimental.pallas import tpu_sc as plsc

N = 10000
NPAD = 10240
D_IN = 128
D_HID = 256
W = 128
CHUNK = 128
NPT = NPAD // 16
NBLK = NPT // CHUNK


def _mesh():
    return plsc.VectorSubcoreMesh(core_axis_name="c", subcore_axis_name="s")


def _fill(ref, val):
    v16 = jnp.full((16,), val, jnp.float32)

    def body(i, carry):
        for j in range(W // 16):
            ref[i, pl.ds(j * 16, 16)] = v16
        return carry

    lax.fori_loop(0, CHUNK, body, 0)


def _degree(dst, epad):
    ept = epad // 2 // 16
    nchunks = ept // CHUNK

    @functools.partial(
        pl.kernel,
        mesh=_mesh(),
        out_type=jax.ShapeDtypeStruct((2 * NPAD, W), jnp.float32),
        scratch_types=[
            pltpu.VMEM_SHARED((NPAD, W), jnp.float32),
            pltpu.VMEM((CHUNK, W), jnp.float32),
            pltpu.VMEM((CHUNK,), jnp.int32),
        ],
    )
    def deg_k(dst_hbm, out_hbm, acc_sh, ones_v, idx_v):
        c = lax.axis_index("c")
        s = lax.axis_index("s")
        _fill(ones_v, 0.0)
        for b in range(NBLK):
            pltpu.sync_copy(ones_v, acc_sh.at[pl.ds(s * NPT + b * CHUNK, CHUNK)])
        _fill(ones_v, 1.0)
        plsc.subcore_barrier()
        base = (c * 16 + s) * ept

        def eb(k, carry):
            pltpu.sync_copy(dst_hbm.at[pl.ds(base + k * CHUNK, CHUNK)], idx_v)
            pltpu.sync_copy(ones_v, acc_sh.at[idx_v], add=True)
            return carry

        lax.fori_loop(0, nchunks, eb, 0)
        plsc.subcore_barrier()
        for b in range(NBLK):
            r0 = s * NPT + b * CHUNK
            pltpu.sync_copy(acc_sh.at[pl.ds(r0, CHUNK)],
                            out_hbm.at[pl.ds(c * NPAD + r0, CHUNK)])

    return deg_k(dst)


def _propagate(xs_flat, src, dst, epad, feat_split):
    ept = (epad if feat_split else epad // 2) // 16
    nchunks = ept // CHUNK
    npairs = nchunks // 2

    @functools.partial(
        pl.kernel,
        mesh=_mesh(),
        out_type=jax.ShapeDtypeStruct((2 * NPAD, W), jnp.float32),
        scratch_types=[
            pltpu.VMEM_SHARED((NPAD, W), jnp.float32),
            pltpu.VMEM((CHUNK, W), jnp.float32),
            pltpu.VMEM((CHUNK, W), jnp.float32),
            pltpu.VMEM((CHUNK,), jnp.int32),
            pltpu.VMEM((CHUNK,), jnp.int32),
            pltpu.VMEM((CHUNK,), jnp.int32),
            pltpu.VMEM((CHUNK,), jnp.int32),
            pltpu.VMEM((CHUNK,), jnp.int32),
            pltpu.VMEM((CHUNK,), jnp.int32),
            pltpu.SemaphoreType.DMA,
            pltpu.SemaphoreType.DMA,
        ],
    )
    def prop_k(xs_hbm, src_hbm, dst_hbm, out_hbm,
               acc_sh, rows0_v, rows1_v, sidx0_v, sidx1_v,
               didx0_v, didx1_v, soff0_v, soff1_v, sem0, sem1):
        c = lax.axis_index("c")
        s = lax.axis_index("s")
        rows = (rows0_v, rows1_v)
        sidx = (sidx0_v, sidx1_v)
        didx = (didx0_v, didx1_v)
        soff = (soff0_v, soff1_v)
        sems = (sem0, sem1)
        _fill(rows0_v, 0.0)
        for b in range(NBLK):
            pltpu.sync_copy(rows0_v, acc_sh.at[pl.ds(s * NPT + b * CHUNK, CHUNK)])
        plsc.subcore_barrier()

        if feat_split:
            off = c * NPAD
            base = s * ept
        else:
            off = c * 0
            base = (c * 16 + s) * ept

        def fire(k, b):
            e0 = base + k * CHUNK
            pltpu.sync_copy(src_hbm.at[pl.ds(e0, CHUNK)], sidx[b])
            pltpu.sync_copy(dst_hbm.at[pl.ds(e0, CHUNK)], didx[b])

            def ob(i, c2):
                soff[b][pl.ds(i * 16, 16)] = sidx[b][pl.ds(i * 16, 16)] + off
                return c2

            lax.fori_loop(0, CHUNK // 16, ob, 0)
            return pltpu.async_copy(xs_hbm.at[soff[b]], rows[b], sems[b])

        def drain(b):
            pltpu.make_async_copy(xs_hbm.at[soff[b]], rows[b], sems[b]).wait()
            pltpu.sync_copy(rows[b], acc_sh.at[didx[b]], add=True)

        fire(0, 0)

        def pair_body(p, carry):
            k0 = 2 * p
            fire(k0 + 1, 1)
            drain(0)
            fire(k0 + 2, 0)
            drain(1)
            return carry

        lax.fori_loop(0, npairs - 1, pair_body, 0)
        fire(nchunks - 1, 1)
        drain(0)
        drain(1)
        plsc.subcore_barrier()
        for b in range(NBLK):
            r0 = s * NPT + b * CHUNK
            pltpu.sync_copy(acc_sh.at[pl.ds(r0, CHUNK)],
                            out_hbm.at[pl.ds(c * NPAD + r0, CHUNK)])

    return prop_k(xs_flat, src, dst)


def _scale(deg_parts, x_pad):
    RB = 2048

    def body(dp_ref, x_ref, xs_ref, dinv_ref):
        deg = dp_ref[0, :, 0:1] + dp_ref[1, :, 0:1] + 1.0
        dinv = lax.rsqrt(deg)
        dinv_ref[...] = dinv
        xs_ref[...] = x_ref[...] * dinv

    return pl.pallas_call(
        body,
        grid=(NPAD // RB,),
        in_specs=[
            pl.BlockSpec((2, RB, W), lambda r: (0, r, 0)),
            pl.BlockSpec((RB, D_IN), lambda r: (r, 0)),
        ],
        out_specs=[
            pl.BlockSpec((RB, D_IN), lambda r: (r, 0)),
            pl.BlockSpec((RB, 1), lambda r: (r, 0)),
        ],
        out_shape=[
            jax.ShapeDtypeStruct((NPAD, D_IN), jnp.float32),
            jax.ShapeDtypeStruct((NPAD, 1), jnp.float32),
        ],
    )(deg_parts, x_pad)


def _layer1(s1, xs, dinv, W1, b1):
    RB = 2048

    def body(s1_ref, xs_ref, dinv_ref, w_ref, b_ref, out_ref):
        dv = dinv_ref[...]
        t = (s1_ref[0] + s1_ref[1] + xs_ref[...]) * dv
        h = jnp.dot(t, w_ref[...], preferred_element_type=jnp.float32)
        h = jnp.maximum(h + b_ref[...], 0.0) * dv
        out_ref[0] = h[:, : D_HID // 2]
        out_ref[1] = h[:, D_HID // 2:]

    return pl.pallas_call(
        body,
        grid=(NPAD // RB,),
        in_specs=[
            pl.BlockSpec((2, RB, D_IN), lambda r: (0, r, 0)),
            pl.BlockSpec((RB, D_IN), lambda r: (r, 0)),
            pl.BlockSpec((RB, 1), lambda r: (r, 0)),
            pl.BlockSpec((D_IN, D_HID), lambda r: (0, 0)),
            pl.BlockSpec((1, D_HID), lambda r: (0, 0)),
        ],
        out_specs=pl.BlockSpec((2, RB, D_HID // 2), lambda r: (0, r, 0)),
        out_shape=jax.ShapeDtypeStruct((2, NPAD, D_HID // 2), jnp.float32),
    )(s1, xs, dinv, W1, b1.reshape(1, D_HID))


def _layer2(s2, hs, dinv, W2, b2):
    RB = 2048

    def body(s2_ref, hs_ref, dinv_ref, w_ref, b_ref, out_ref):
        dv = dinv_ref[...]
        t = jnp.concatenate(
            [(s2_ref[0] + hs_ref[0]) * dv, (s2_ref[1] + hs_ref[1]) * dv],
            axis=1)
        out_ref[...] = (
            jnp.dot(t, w_ref[...], preferred_element_type=jnp.float32)
            + b_ref[...])

    return pl.pallas_call(
        body,
        grid=(NPAD // RB,),
        in_specs=[
            pl.BlockSpec((2, RB, D_HID // 2), lambda r: (0, r, 0)),
            pl.BlockSpec((2, RB, D_HID // 2), lambda r: (0, r, 0)),
            pl.BlockSpec((RB, 1), lambda r: (r, 0)),
            pl.BlockSpec((D_HID, D_HID), lambda r: (0, 0)),
            pl.BlockSpec((1, D_HID), lambda r: (0, 0)),
        ],
        out_specs=pl.BlockSpec((RB, D_HID), lambda r: (r, 0)),
        out_shape=jax.ShapeDtypeStruct((NPAD, D_HID), jnp.float32),
    )(s2, hs, dinv, W2, b2.reshape(1, D_HID))


def kernel(x, edge_index, W1, b1, W2, b2):
    e = edge_index.shape[1]
    step = 64 * CHUNK
    epad = ((e + step - 1) // step) * step
    ei = edge_index.astype(jnp.int32)
    pad = jnp.full((epad - e,), N, jnp.int32)
    src = jnp.concatenate([ei[0], pad])
    dst = jnp.concatenate([ei[1], pad])
    x_pad = jnp.pad(x, ((0, NPAD - N), (0, 0)))

    deg_parts = _degree(dst, epad).reshape(2, NPAD, W)
    xs, dinv = _scale(deg_parts, x_pad)
    s1 = _propagate(xs, src, dst, epad,
                    feat_split=False).reshape(2, NPAD, D_IN)
    hs = _layer1(s1, xs, dinv, W1, b1)
    s2 = _propagate(hs.reshape(2 * NPAD, D_HID // 2), src, dst, epad,
                    feat_split=True).reshape(2, NPAD, D_HID // 2)
    z = _layer2(s2, hs, dinv, W2, b2)
    return z[:N]


# speedup vs baseline: 1.0248x; 1.0248x over previous
import functools

import jax
import jax.numpy as jnp
from jax import lax
from jax.experimental import pallas as pl
from jax.experimental.pallas import tpu as pltpu
from jax.experimental.pallas import tpu_sc as plsc

N = 10000
NPAD = 10240
D_IN = 128
D_HID = 256
W = 128
CHUNK = 128
NPT = NPAD // 16
NBLK = NPT // CHUNK


def _mesh():
    return plsc.VectorSubcoreMesh(core_axis_name="c", subcore_axis_name="s")


def _fill(ref, val):
    v16 = jnp.full((16,), val, jnp.float32)

    def body(i, carry):
        for j in range(W // 16):
            ref[i, pl.ds(j * 16, 16)] = v16
        return carry

    lax.fori_loop(0, CHUNK, body, 0)


KB = 8


def _degree(eidx, epad):
    nct = epad // 2 // 16 // CHUNK
    nbt = nct // KB

    @functools.partial(
        pl.kernel,
        mesh=_mesh(),
        out_type=jax.ShapeDtypeStruct((2 * NPAD, W), jnp.float32),
        scratch_types=[
            pltpu.VMEM_SHARED((NPAD, W), jnp.float32),
            pltpu.VMEM((CHUNK, W), jnp.float32),
            pltpu.VMEM((2 * KB, CHUNK), jnp.int32),
        ],
    )
    def deg_k(eidx_hbm, out_hbm, acc_sh, ones_v, idxb_v):
        c = lax.axis_index("c")
        s = lax.axis_index("s")
        _fill(ones_v, 0.0)
        for b in range(NBLK):
            pltpu.sync_copy(ones_v, acc_sh.at[pl.ds(s * NPT + b * CHUNK, CHUNK)])
        _fill(ones_v, 1.0)
        plsc.subcore_barrier()
        tb = (c * 16 + s) * nct

        def bb(i, carry):
            pltpu.sync_copy(eidx_hbm.at[pl.ds(2 * (tb + i * KB), 2 * KB)],
                            idxb_v)
            for j in range(KB):
                pltpu.sync_copy(ones_v, acc_sh.at[idxb_v.at[2 * j + 1]],
                                add=True)
            return carry

        lax.fori_loop(0, nbt, bb, 0)
        plsc.subcore_barrier()
        for b in range(NBLK):
            r0 = s * NPT + b * CHUNK
            pltpu.sync_copy(acc_sh.at[pl.ds(r0, CHUNK)],
                            out_hbm.at[pl.ds(c * NPAD + r0, CHUNK)])

    return deg_k(eidx)


def _propagate(xs_flat, eidx, epad, feat_split):
    nct = (epad if feat_split else epad // 2) // 16 // CHUNK
    npairs = nct // KB // 2

    @functools.partial(
        pl.kernel,
        mesh=_mesh(),
        out_type=jax.ShapeDtypeStruct((2 * NPAD, W), jnp.float32),
        scratch_types=[
            pltpu.VMEM_SHARED((NPAD, W), jnp.float32),
            pltpu.VMEM((CHUNK, W), jnp.float32),
            pltpu.VMEM((CHUNK, W), jnp.float32),
            pltpu.VMEM((2 * KB, CHUNK), jnp.int32),
            pltpu.VMEM((2 * KB, CHUNK), jnp.int32),
            pltpu.VMEM((CHUNK,), jnp.int32),
            pltpu.VMEM((CHUNK,), jnp.int32),
            pltpu.SemaphoreType.DMA,
            pltpu.SemaphoreType.DMA,
        ],
    )
    def prop_k(xs_hbm, eidx_hbm, out_hbm,
               acc_sh, rows0_v, rows1_v, idxb0_v, idxb1_v,
               soff0_v, soff1_v, sem0, sem1):
        c = lax.axis_index("c")
        s = lax.axis_index("s")
        rows = (rows0_v, rows1_v)
        idxb = (idxb0_v, idxb1_v)
        soff = (soff0_v, soff1_v)
        sems = (sem0, sem1)
        _fill(rows0_v, 0.0)
        for b in range(NBLK):
            pltpu.sync_copy(rows0_v, acc_sh.at[pl.ds(s * NPT + b * CHUNK, CHUNK)])
        plsc.subcore_barrier()

        off = c * NPAD if feat_split else None
        tb = (s if feat_split else c * 16 + s) * nct

        def load_block(i, ib):
            pltpu.sync_copy(eidx_hbm.at[pl.ds(2 * (tb + i * KB), 2 * KB)],
                            idxb[ib])

        def src_ref(ib, j, b):
            if feat_split:
                return xs_hbm.at[soff[b]]
            return xs_hbm.at[idxb[ib].at[2 * j]]

        def fire(ib, j, b):
            if feat_split:
                for i in range(CHUNK // 16):
                    sl = pl.ds(i * 16, 16)
                    soff[b][sl] = idxb[ib][2 * j, sl] + off
            pltpu.async_copy(src_ref(ib, j, b), rows[b], sems[b])

        def drain(ib, j, b):
            pltpu.make_async_copy(src_ref(ib, j, b), rows[b], sems[b]).wait()
            pltpu.sync_copy(rows[b], acc_sh.at[idxb[ib].at[2 * j + 1]],
                            add=True)

        seq = [(0, j) for j in range(KB)] + [(1, j) for j in range(KB)]

        load_block(0, 0)

        def pair_body(p, carry):
            sb = 2 * p
            fire(0, 0, 0)
            fire(0, 1, 1)
            load_block(sb + 1, 1)
            for t in range(2, 2 * KB):
                ibp, jp = seq[t - 2]
                drain(ibp, jp, t % 2)
                fire(*seq[t], t % 2)
            load_block(sb + 2, 0)
            drain(*seq[2 * KB - 2], 0)
            drain(*seq[2 * KB - 1], 1)
            return carry

        lax.fori_loop(0, npairs, pair_body, 0)
        plsc.subcore_barrier()
        for b in range(NBLK):
            r0 = s * NPT + b * CHUNK
            pltpu.sync_copy(acc_sh.at[pl.ds(r0, CHUNK)],
                            out_hbm.at[pl.ds(c * NPAD + r0, CHUNK)])

    return prop_k(xs_flat, eidx)


def _scale(deg_parts, x_pad):
    RB = 2048

    def body(dp_ref, x_ref, xs_ref, dinv_ref):
        deg = dp_ref[0, :, 0:1] + dp_ref[1, :, 0:1] + 1.0
        dinv = lax.rsqrt(deg)
        dinv_ref[...] = dinv
        xs_ref[...] = x_ref[...] * dinv

    return pl.pallas_call(
        body,
        grid=(NPAD // RB,),
        in_specs=[
            pl.BlockSpec((2, RB, W), lambda r: (0, r, 0)),
            pl.BlockSpec((RB, D_IN), lambda r: (r, 0)),
        ],
        out_specs=[
            pl.BlockSpec((RB, D_IN), lambda r: (r, 0)),
            pl.BlockSpec((RB, 1), lambda r: (r, 0)),
        ],
        out_shape=[
            jax.ShapeDtypeStruct((NPAD, D_IN), jnp.float32),
            jax.ShapeDtypeStruct((NPAD, 1), jnp.float32),
        ],
    )(deg_parts, x_pad)


def _layer1(s1, xs, dinv, W1, b1):
    RB = 2048

    def body(s1_ref, xs_ref, dinv_ref, w_ref, b_ref, out_ref):
        dv = dinv_ref[...]
        t = (s1_ref[0] + s1_ref[1] + xs_ref[...]) * dv
        h = jnp.dot(t, w_ref[...], preferred_element_type=jnp.float32)
        h = jnp.maximum(h + b_ref[...], 0.0) * dv
        out_ref[0] = h[:, : D_HID // 2]
        out_ref[1] = h[:, D_HID // 2:]

    return pl.pallas_call(
        body,
        grid=(NPAD // RB,),
        in_specs=[
            pl.BlockSpec((2, RB, D_IN), lambda r: (0, r, 0)),
            pl.BlockSpec((RB, D_IN), lambda r: (r, 0)),
            pl.BlockSpec((RB, 1), lambda r: (r, 0)),
            pl.BlockSpec((D_IN, D_HID), lambda r: (0, 0)),
            pl.BlockSpec((1, D_HID), lambda r: (0, 0)),
        ],
        out_specs=pl.BlockSpec((2, RB, D_HID // 2), lambda r: (0, r, 0)),
        out_shape=jax.ShapeDtypeStruct((2, NPAD, D_HID // 2), jnp.float32),
    )(s1, xs, dinv, W1, b1.reshape(1, D_HID))


def _layer2(s2, hs, dinv, W2, b2):
    RB = 2048

    def body(s2_ref, hs_ref, dinv_ref, w_ref, b_ref, out_ref):
        dv = dinv_ref[...]
        t = jnp.concatenate(
            [(s2_ref[0] + hs_ref[0]) * dv, (s2_ref[1] + hs_ref[1]) * dv],
            axis=1)
        out_ref[...] = (
            jnp.dot(t, w_ref[...], preferred_element_type=jnp.float32)
            + b_ref[...])

    return pl.pallas_call(
        body,
        grid=(NPAD // RB,),
        in_specs=[
            pl.BlockSpec((2, RB, D_HID // 2), lambda r: (0, r, 0)),
            pl.BlockSpec((2, RB, D_HID // 2), lambda r: (0, r, 0)),
            pl.BlockSpec((RB, 1), lambda r: (r, 0)),
            pl.BlockSpec((D_HID, D_HID), lambda r: (0, 0)),
            pl.BlockSpec((1, D_HID), lambda r: (0, 0)),
        ],
        out_specs=pl.BlockSpec((RB, D_HID), lambda r: (r, 0)),
        out_shape=jax.ShapeDtypeStruct((NPAD, D_HID), jnp.float32),
    )(s2, hs, dinv, W2, b2.reshape(1, D_HID))


def kernel(x, edge_index, W1, b1, W2, b2):
    e = edge_index.shape[1]
    step = 64 * CHUNK
    epad = ((e + step - 1) // step) * step
    ei = edge_index.astype(jnp.int32)
    pad = jnp.full((epad - e,), N, jnp.int32)
    src = jnp.concatenate([ei[0], pad]).reshape(-1, CHUNK)
    dst = jnp.concatenate([ei[1], pad]).reshape(-1, CHUNK)
    eidx = jnp.concatenate([
        jnp.stack([src, dst], axis=1).reshape(-1, CHUNK),
        jnp.full((2 * KB, CHUNK), N, jnp.int32),
    ])
    x_pad = jnp.pad(x, ((0, NPAD - N), (0, 0)))

    deg_parts = _degree(eidx, epad).reshape(2, NPAD, W)
    xs, dinv = _scale(deg_parts, x_pad)
    s1 = _propagate(xs, eidx, epad,
                    feat_split=False).reshape(2, NPAD, D_IN)
    hs = _layer1(s1, xs, dinv, W1, b1)
    s2 = _propagate(hs.reshape(2 * NPAD, D_HID // 2), eidx, epad,
                    feat_split=True).reshape(2, NPAD, D_HID // 2)
    z = _layer2(s2, hs, dinv, W2, b2)
    return z[:N]


# speedup vs baseline: 1.0260x; 1.0012x over previous
import functools

import jax
import jax.numpy as jnp
from jax import lax
from jax.experimental import pallas as pl
from jax.experimental.pallas import tpu as pltpu
from jax.experimental.pallas import tpu_sc as plsc

N = 10000
NPAD = 10240
D_IN = 128
D_HID = 256
W = 128
CHUNK = 128
NPT = NPAD // 16
NBLK = NPT // CHUNK


def _mesh():
    return plsc.VectorSubcoreMesh(core_axis_name="c", subcore_axis_name="s")


def _fill(ref, val):
    v16 = jnp.full((16,), val, jnp.float32)

    def body(i, carry):
        for j in range(W // 16):
            ref[i, pl.ds(j * 16, 16)] = v16
        return carry

    lax.fori_loop(0, CHUNK, body, 0)


KB = 8


def _degree(eidx, epad):
    nct = epad // 2 // 16 // CHUNK
    nbt = nct // KB

    @functools.partial(
        pl.kernel,
        mesh=_mesh(),
        out_type=jax.ShapeDtypeStruct((2 * NPAD, W), jnp.float32),
        scratch_types=[
            pltpu.VMEM_SHARED((NPAD, W), jnp.float32),
            pltpu.VMEM((CHUNK, W), jnp.float32),
            pltpu.VMEM((2 * KB, CHUNK), jnp.int32),
        ],
    )
    def deg_k(eidx_hbm, out_hbm, acc_sh, ones_v, idxb_v):
        c = lax.axis_index("c")
        s = lax.axis_index("s")
        _fill(ones_v, 0.0)
        for b in range(NBLK):
            pltpu.sync_copy(ones_v, acc_sh.at[pl.ds(s * NPT + b * CHUNK, CHUNK)])
        _fill(ones_v, 1.0)
        plsc.subcore_barrier()
        tb = (c * 16 + s) * nct

        def bb(i, carry):
            pltpu.sync_copy(eidx_hbm.at[pl.ds(2 * (tb + i * KB), 2 * KB)],
                            idxb_v)
            for j in range(KB):
                pltpu.sync_copy(ones_v, acc_sh.at[idxb_v.at[2 * j + 1]],
                                add=True)
            return carry

        lax.fori_loop(0, nbt, bb, 0)
        plsc.subcore_barrier()
        for b in range(NBLK):
            r0 = s * NPT + b * CHUNK
            pltpu.sync_copy(acc_sh.at[pl.ds(r0, CHUNK)],
                            out_hbm.at[pl.ds(c * NPAD + r0, CHUNK)])

    return deg_k(eidx)


def _propagate(xs_flat, eidx, epad, feat_split):
    nct = (epad if feat_split else epad // 2) // 16 // CHUNK
    npairs = nct // KB // 2

    @functools.partial(
        pl.kernel,
        mesh=_mesh(),
        out_type=jax.ShapeDtypeStruct((2 * NPAD, W), jnp.float32),
        scratch_types=[
            pltpu.VMEM_SHARED((NPAD, W), jnp.float32),
            pltpu.VMEM((2, CHUNK, W), jnp.float32),
            pltpu.VMEM((2 * KB, CHUNK), jnp.int32),
            pltpu.VMEM((2 * KB, CHUNK), jnp.int32),
            pltpu.VMEM((2, CHUNK), jnp.int32),
            pltpu.SemaphoreType.DMA,
            pltpu.SemaphoreType.DMA,
            pltpu.SemaphoreType.DMA,
            pltpu.SemaphoreType.DMA,
        ],
    )
    def prop_k(xs_hbm, eidx_hbm, out_hbm,
               acc_sh, rows_v, idxb0_v, idxb1_v, soff_v,
               g0, g1, s0, s1):
        c = lax.axis_index("c")
        s = lax.axis_index("s")
        idxb = (idxb0_v, idxb1_v)
        gsem = (g0, g1)
        ssem = (s0, s1)
        _fill(rows_v.at[0], 0.0)
        for b in range(NBLK):
            pltpu.sync_copy(rows_v.at[0],
                            acc_sh.at[pl.ds(s * NPT + b * CHUNK, CHUNK)])
        plsc.subcore_barrier()

        off = c * NPAD if feat_split else None
        tb = (s if feat_split else c * 16 + s) * nct

        def load_block(i, ib):
            pltpu.sync_copy(eidx_hbm.at[pl.ds(2 * (tb + i * KB), 2 * KB)],
                            idxb[ib])

        def src_ref(ib, j, b):
            if feat_split:
                return xs_hbm.at[soff_v.at[b]]
            return xs_hbm.at[idxb[ib].at[2 * j]]

        def fire_g(ib, j, b):
            if feat_split:
                for i in range(CHUNK // 16):
                    sl = pl.ds(i * 16, 16)
                    soff_v[b, sl] = idxb[ib][2 * j, sl] + off
            pltpu.async_copy(src_ref(ib, j, b), rows_v.at[b], gsem[b])

        def wait_g(ib, j, b):
            pltpu.make_async_copy(src_ref(ib, j, b), rows_v.at[b],
                                  gsem[b]).wait()

        def fire_s(ib, j, b):
            pltpu.async_copy(rows_v.at[b], acc_sh.at[idxb[ib].at[2 * j + 1]],
                             add=True, sem=ssem[b])

        def wait_s(b):
            pltpu.make_async_copy(rows_v.at[b], acc_sh.at[idxb[0].at[1]],
                                  ssem[b]).wait()

        seq = [(0, j) for j in range(KB)] + [(1, j) for j in range(KB)]

        load_block(0, 0)

        def pair_body(p, carry):
            sb = 2 * p
            for t in range(2 * KB):
                b = t % 2
                if t >= 2:
                    wait_s(b)
                fire_g(*seq[t], b)
                if t == 1:
                    load_block(sb + 1, 1)
                if t >= 1:
                    wait_g(*seq[t - 1], 1 - b)
                    fire_s(*seq[t - 1], 1 - b)
            load_block(sb + 2, 0)
            wait_g(*seq[2 * KB - 1], 1)
            fire_s(*seq[2 * KB - 1], 1)
            wait_s(0)
            wait_s(1)
            return carry

        lax.fori_loop(0, npairs, pair_body, 0)
        plsc.subcore_barrier()
        for b in range(NBLK):
            r0 = s * NPT + b * CHUNK
            pltpu.sync_copy(acc_sh.at[pl.ds(r0, CHUNK)],
                            out_hbm.at[pl.ds(c * NPAD + r0, CHUNK)])

    return prop_k(xs_flat, eidx)


def _scale(deg_parts, x_pad):
    RB = 2048

    def body(dp_ref, x_ref, xs_ref, dinv_ref):
        deg = dp_ref[0, :, 0:1] + dp_ref[1, :, 0:1] + 1.0
        dinv = lax.rsqrt(deg)
        dinv_ref[...] = dinv
        xs_ref[...] = x_ref[...] * dinv

    return pl.pallas_call(
        body,
        grid=(NPAD // RB,),
        in_specs=[
            pl.BlockSpec((2, RB, W), lambda r: (0, r, 0)),
            pl.BlockSpec((RB, D_IN), lambda r: (r, 0)),
        ],
        out_specs=[
            pl.BlockSpec((RB, D_IN), lambda r: (r, 0)),
            pl.BlockSpec((RB, 1), lambda r: (r, 0)),
        ],
        out_shape=[
            jax.ShapeDtypeStruct((NPAD, D_IN), jnp.float32),
            jax.ShapeDtypeStruct((NPAD, 1), jnp.float32),
        ],
    )(deg_parts, x_pad)


def _layer1(s1, xs, dinv, W1, b1):
    RB = 2048

    def body(s1_ref, xs_ref, dinv_ref, w_ref, b_ref, out_ref):
        dv = dinv_ref[...]
        t = (s1_ref[0] + s1_ref[1] + xs_ref[...]) * dv
        h = jnp.dot(t, w_ref[...], preferred_element_type=jnp.float32)
        h = jnp.maximum(h + b_ref[...], 0.0) * dv
        out_ref[0] = h[:, : D_HID // 2]
        out_ref[1] = h[:, D_HID // 2:]

    return pl.pallas_call(
        body,
        grid=(NPAD // RB,),
        in_specs=[
            pl.BlockSpec((2, RB, D_IN), lambda r: (0, r, 0)),
            pl.BlockSpec((RB, D_IN), lambda r: (r, 0)),
            pl.BlockSpec((RB, 1), lambda r: (r, 0)),
            pl.BlockSpec((D_IN, D_HID), lambda r: (0, 0)),
            pl.BlockSpec((1, D_HID), lambda r: (0, 0)),
        ],
        out_specs=pl.BlockSpec((2, RB, D_HID // 2), lambda r: (0, r, 0)),
        out_shape=jax.ShapeDtypeStruct((2, NPAD, D_HID // 2), jnp.float32),
    )(s1, xs, dinv, W1, b1.reshape(1, D_HID))


def _layer2(s2, hs, dinv, W2, b2):
    RB = 2048

    def body(s2_ref, hs_ref, dinv_ref, w_ref, b_ref, out_ref):
        dv = dinv_ref[...]
        t = jnp.concatenate(
            [(s2_ref[0] + hs_ref[0]) * dv, (s2_ref[1] + hs_ref[1]) * dv],
            axis=1)
        out_ref[...] = (
            jnp.dot(t, w_ref[...], preferred_element_type=jnp.float32)
            + b_ref[...])

    return pl.pallas_call(
        body,
        grid=(NPAD // RB,),
        in_specs=[
            pl.BlockSpec((2, RB, D_HID // 2), lambda r: (0, r, 0)),
            pl.BlockSpec((2, RB, D_HID // 2), lambda r: (0, r, 0)),
            pl.BlockSpec((RB, 1), lambda r: (r, 0)),
            pl.BlockSpec((D_HID, D_HID), lambda r: (0, 0)),
            pl.BlockSpec((1, D_HID), lambda r: (0, 0)),
        ],
        out_specs=pl.BlockSpec((RB, D_HID), lambda r: (r, 0)),
        out_shape=jax.ShapeDtypeStruct((NPAD, D_HID), jnp.float32),
    )(s2, hs, dinv, W2, b2.reshape(1, D_HID))


def kernel(x, edge_index, W1, b1, W2, b2):
    e = edge_index.shape[1]
    step = 64 * CHUNK
    epad = ((e + step - 1) // step) * step
    ei = edge_index.astype(jnp.int32)
    pad = jnp.full((epad - e,), N, jnp.int32)
    src = jnp.concatenate([ei[0], pad]).reshape(-1, CHUNK)
    dst = jnp.concatenate([ei[1], pad]).reshape(-1, CHUNK)
    eidx = jnp.concatenate([
        jnp.stack([src, dst], axis=1).reshape(-1, CHUNK),
        jnp.full((2 * KB, CHUNK), N, jnp.int32),
    ])
    x_pad = jnp.pad(x, ((0, NPAD - N), (0, 0)))

    deg_parts = _degree(eidx, epad).reshape(2, NPAD, W)
    xs, dinv = _scale(deg_parts, x_pad)
    s1 = _propagate(xs, eidx, epad,
                    feat_split=False).reshape(2, NPAD, D_IN)
    hs = _layer1(s1, xs, dinv, W1, b1)
    s2 = _propagate(hs.reshape(2 * NPAD, D_HID // 2), eidx, epad,
                    feat_split=True).reshape(2, NPAD, D_HID // 2)
    z = _layer2(s2, hs, dinv, W2, b2)
    return z[:N]
